# baseline (speedup 1.0000x reference)
import functools

import jax
import jax.numpy as jnp
from jax import lax
from jax.experimental import pallas as pl
from jax.experimental.pallas import tpu as pltpu
from jax.experimental.pallas import tpu_sc as plsc

D_MODEL = 64
D_PAD = 128
B_TOTAL = 16384 * 20
NUM_WORKERS = 32
CHUNK = 128
K = 8
KH = 4
ROWS_PER_W = B_TOTAL // (NUM_WORKERS * CHUNK)
NUM_BLOCKS = ROWS_PER_W // K

_mesh = plsc.VectorSubcoreMesh(core_axis_name="c", subcore_axis_name="s")


@functools.partial(
    pl.kernel,
    mesh=_mesh,
    out_type=jax.ShapeDtypeStruct((B_TOTAL // CHUNK, CHUNK, D_PAD),
                                  jnp.float32),
    scratch_types=[
        pltpu.VMEM((K, CHUNK), jnp.int32),
        pltpu.VMEM((KH, CHUNK, D_PAD), jnp.float32),
        pltpu.SemaphoreType.DMA,
    ],
)
def _gather_kernel(idx_hbm, table_hbm, out_hbm, idx_v, rows_v, sem):
    wid = lax.axis_index("s") * 2 + lax.axis_index("c")
    base_row = wid * ROWS_PER_W

    def body(blk, carry):
        row = base_row + blk * K
        pltpu.sync_copy(idx_hbm.at[pl.ds(row, K)], idx_v)
        for h in range(K // KH):
            copies = [
                pltpu.async_copy(table_hbm.at[idx_v.at[h * KH + j]],
                                 rows_v.at[j], sem)
                for j in range(KH)
            ]
            for c in copies:
                c.wait()
            pltpu.sync_copy(rows_v, out_hbm.at[pl.ds(row + h * KH, KH)])
        return carry

    lax.fori_loop(0, NUM_BLOCKS, body, 0)


def kernel(token_ids, weight):
    idx = token_ids.reshape(B_TOTAL // CHUNK, CHUNK).astype(jnp.int32)
    wp = jnp.pad(weight, ((0, 0), (0, D_PAD - D_MODEL)))
    out = _gather_kernel(idx, wp)
    return out[:, :, :D_MODEL].reshape(token_ids.shape + (D_MODEL,))
